# baseline (speedup 1.0000x reference)
import functools

import jax
import jax.numpy as jnp
from jax.experimental import pallas as pl
from jax.experimental.pallas import tpu as pltpu

_K = 1024
_D = 128
_ROT = ((13, 15, 26, 6), (17, 29, 16, 24))


def _threefry(x0, x1, k0, k1):
    k2 = k0 ^ k1 ^ jnp.uint32(0x1BD11BDA)
    ks = (k0, k1, k2)
    x0 = x0 + k0
    x1 = x1 + k1
    for i in range(5):
        for r in _ROT[i % 2]:
            x0 = x0 + x1
            x1 = (x1 << r) | (x1 >> (32 - r))
            x1 = x1 ^ x0
        x0 = x0 + ks[(i + 1) % 3]
        x1 = x1 + ks[(i + 2) % 3] + jnp.uint32(i + 1)
    return x0, x1


def _bits_to_unit(bits_f32_mantissa):
    return bits_f32_mantissa * jnp.float32(2.0 ** -23)


def _erfinv_f32(x):
    w = -jnp.log1p(-x * x)
    w_c = w - jnp.float32(2.5)
    p_c = jnp.float32(2.81022636e-08)
    for c in (3.43273939e-07, -3.5233877e-06, -4.39150654e-06, 0.00021858087,
              -0.00125372503, -0.00417768164, 0.246640727, 1.50140941):
        p_c = jnp.float32(c) + p_c * w_c
    w_t = jnp.sqrt(w) - jnp.float32(3.0)
    p_t = jnp.float32(-0.000200214257)
    for c in (0.000100950558, 0.00134934322, -0.00367342844, 0.00573950773,
              -0.0076224613, 0.00943887047, 1.00167406, 2.83297682):
        p_t = jnp.float32(c) + p_t * w_t
    return jnp.where(w < jnp.float32(5.0), p_c, p_t) * x


def _body(kd_ref, w_ref, means_hbm, covs_hbm, out_ref, acov, mrow, sem_c, sem_m):
    u32 = jnp.uint32
    k0 = kd_ref[0]
    k1 = kd_ref[1]

    a0, b0 = _threefry(u32(0), u32(0), k0, k1)
    a1, b1 = _threefry(u32(0), u32(1), k0, k1)
    ki0, ki1 = a0, b0
    ks0, ks1 = a1, b1

    ua, ub = _threefry(u32(0), u32(0), ki0, ki1)
    ubits = ua ^ ub
    u = _bits_to_unit(((ubits >> 9)).astype(jnp.int32).astype(jnp.float32))

    w = w_ref[:]
    s_total = jnp.sum(w)
    p = w / s_total
    lane = jax.lax.broadcasted_iota(jnp.int32, (_D, _D), 0)
    lane_t = jax.lax.broadcasted_iota(jnp.int32, (_D, _D), 1)
    triu = (lane <= lane_t).astype(jnp.float32)
    lanecum = jnp.dot(p, triu, preferred_element_type=jnp.float32)
    rowtot = lanecum[:, _D - 1:_D]
    inc = rowtot
    for d in (1, 2, 4):
        inc = inc + jnp.concatenate(
            [jnp.zeros((d, 1), jnp.float32), inc[:-d]], axis=0)
    offs = jnp.concatenate(
        [jnp.zeros((1, 1), jnp.float32), inc[:-1]], axis=0)
    p_cuml = offs + lanecum
    r = jnp.max(p_cuml) * (jnp.float32(1.0) - u)
    idx = jnp.sum((p_cuml < r).astype(jnp.int32))

    cov_copy = pltpu.make_async_copy(covs_hbm.at[idx], acov, sem_c)
    cov_copy.start()
    mean_copy = pltpu.make_async_copy(means_hbm.at[pl.ds(idx, 1), :], mrow, sem_m)
    mean_copy.start()

    cnt = jax.lax.broadcasted_iota(jnp.uint32, (1, _D), 1)
    na, nb = _threefry(jnp.zeros((1, _D), jnp.uint32), cnt, ks0, ks1)
    nbits = na ^ nb
    u01 = _bits_to_unit(((nbits >> 9)).astype(jnp.int32).astype(jnp.float32))
    lo = jnp.float32(-0.99999994)
    un = jnp.maximum(lo, u01 * (jnp.float32(1.0) - lo) + lo)
    z = jnp.float32(1.4142135623730951) * _erfinv_f32(un)

    cov_copy.wait()
    mean_copy.wait()

    lane_row = jax.lax.broadcasted_iota(jnp.int32, (1, _D), 1)
    _PW = 16
    sub16 = jax.lax.broadcasted_iota(jnp.int32, (_PW, 1), 0)
    oh_sub = jax.lax.broadcasted_iota(jnp.int32, (_D, _PW), 0)
    oh_lane = jax.lax.broadcasted_iota(jnp.int32, (_D, _PW), 1)
    eye_cols = [(sub16 == s).astype(jnp.float32) for s in range(_PW)]

    def panel_step(p, y):
        base = _PW * p
        slab = acov[pl.ds(base, _PW), :]
        ohp = (oh_sub == base + oh_lane).astype(jnp.float32)
        cc = jax.lax.dot_general(
            slab, ohp, (((1,), (0,)), ((), ())),
            preferred_element_type=jnp.float32)
        zp = jax.lax.dot_general(
            ohp, z, (((0,), (1,)), ((), ())),
            preferred_element_type=jnp.float32)
        cols = [cc[:, s:s + 1] for s in range(_PW)]
        icols = list(eye_cols)
        ms_cols = []
        for t in range(_PW):
            colv = cols[t]
            pivot = colv[t:t + 1, :]
            ip = jnp.float32(1.0) / pivot
            irs = jax.lax.rsqrt(pivot)
            q = jnp.where(sub16 > t, colv * ip, jnp.float32(0.0))
            for s in range(t + 1, _PW):
                cols[s] = cols[s] - q * colv[s:s + 1, :]
            for s in range(t + 1):
                icols[s] = icols[s] - q * icols[s][t:t + 1, :]
            ms_cols.append(irs)
        irs_col = jnp.concatenate(ms_cols, axis=0)
        m = jnp.concatenate([ic * irs_col for ic in icols], axis=1)
        w = jax.lax.dot_general(
            m, slab, (((1,), (0,)), ((), ())),
            preferred_element_type=jnp.float32)
        wm = jnp.where(lane_row >= base, w, jnp.float32(0.0))
        y = y + jax.lax.dot_general(
            zp, wm, (((0,), (0,)), ((), ())),
            preferred_element_type=jnp.float32)
        u = jax.lax.dot_general(
            wm, wm, (((0,), (0,)), ((), ())),
            preferred_element_type=jnp.float32)
        acov[:] = acov[:] - u
        return y

    y = jax.lax.fori_loop(0, _D // _PW, panel_step,
                          jnp.zeros((1, _D), jnp.float32))
    out_ref[:] = mrow[:] + y


@jax.jit
def kernel(means, covs, weights, key_seed):
    kd = jax.random.key_data(jax.random.key(key_seed)).astype(jnp.uint32)
    out = pl.pallas_call(
        _body,
        out_shape=jax.ShapeDtypeStruct((1, _D), jnp.float32),
        in_specs=[
            pl.BlockSpec(memory_space=pltpu.MemorySpace.SMEM),
            pl.BlockSpec(memory_space=pltpu.MemorySpace.VMEM),
            pl.BlockSpec(memory_space=pl.ANY),
            pl.BlockSpec(memory_space=pl.ANY),
        ],
        out_specs=pl.BlockSpec(memory_space=pltpu.MemorySpace.VMEM),
        scratch_shapes=[
            pltpu.VMEM((_D, _D), jnp.float32),
            pltpu.VMEM((1, _D), jnp.float32),
            pltpu.SemaphoreType.DMA,
            pltpu.SemaphoreType.DMA,
        ],
    )(kd, weights.reshape(8, _D), means, covs)
    return out.reshape(_D)


# speedup vs baseline: 1.6829x; 1.6829x over previous
import functools

import jax
import jax.numpy as jnp
from jax.experimental import pallas as pl
from jax.experimental.pallas import tpu as pltpu

_K = 1024
_D = 128
_ROT = ((13, 15, 26, 6), (17, 29, 16, 24))


def _threefry(x0, x1, k0, k1):
    k2 = k0 ^ k1 ^ jnp.uint32(0x1BD11BDA)
    ks = (k0, k1, k2)
    x0 = x0 + k0
    x1 = x1 + k1
    for i in range(5):
        for r in _ROT[i % 2]:
            x0 = x0 + x1
            x1 = (x1 << r) | (x1 >> (32 - r))
            x1 = x1 ^ x0
        x0 = x0 + ks[(i + 1) % 3]
        x1 = x1 + ks[(i + 2) % 3] + jnp.uint32(i + 1)
    return x0, x1


def _bits_to_unit(bits_f32_mantissa):
    return bits_f32_mantissa * jnp.float32(2.0 ** -23)


def _erfinv_f32(x):
    w = -jnp.log1p(-x * x)
    w_c = w - jnp.float32(2.5)
    p_c = jnp.float32(2.81022636e-08)
    for c in (3.43273939e-07, -3.5233877e-06, -4.39150654e-06, 0.00021858087,
              -0.00125372503, -0.00417768164, 0.246640727, 1.50140941):
        p_c = jnp.float32(c) + p_c * w_c
    w_t = jnp.sqrt(w) - jnp.float32(3.0)
    p_t = jnp.float32(-0.000200214257)
    for c in (0.000100950558, 0.00134934322, -0.00367342844, 0.00573950773,
              -0.0076224613, 0.00943887047, 1.00167406, 2.83297682):
        p_t = jnp.float32(c) + p_t * w_t
    return jnp.where(w < jnp.float32(5.0), p_c, p_t) * x


def _body(kd_ref, w_ref, means_hbm, covs_hbm, out_ref, acov, mrow, sem_c, sem_m):
    u32 = jnp.uint32
    k0 = u32(0)
    k1 = kd_ref[0].astype(jnp.uint32)

    a0, b0 = _threefry(u32(0), u32(0), k0, k1)
    a1, b1 = _threefry(u32(0), u32(1), k0, k1)
    ki0, ki1 = a0, b0
    ks0, ks1 = a1, b1

    ua, ub = _threefry(u32(0), u32(0), ki0, ki1)
    ubits = ua ^ ub
    u = _bits_to_unit(((ubits >> 9)).astype(jnp.int32).astype(jnp.float32))

    w = w_ref[:]
    s_total = jnp.sum(w)
    p = w / s_total
    lane = jax.lax.broadcasted_iota(jnp.int32, (_D, _D), 0)
    lane_t = jax.lax.broadcasted_iota(jnp.int32, (_D, _D), 1)
    triu = (lane <= lane_t).astype(jnp.float32)
    lanecum = jnp.dot(p, triu, preferred_element_type=jnp.float32)
    rowtot = lanecum[:, _D - 1:_D]
    inc = rowtot
    for d in (1, 2, 4):
        inc = inc + jnp.concatenate(
            [jnp.zeros((d, 1), jnp.float32), inc[:-d]], axis=0)
    offs = jnp.concatenate(
        [jnp.zeros((1, 1), jnp.float32), inc[:-1]], axis=0)
    p_cuml = offs + lanecum
    r = jnp.max(p_cuml) * (jnp.float32(1.0) - u)
    idx = jnp.sum((p_cuml < r).astype(jnp.int32))

    cov_copy = pltpu.make_async_copy(covs_hbm.at[idx], acov, sem_c)
    cov_copy.start()
    mean_copy = pltpu.make_async_copy(means_hbm.at[pl.ds(idx, 1), :], mrow, sem_m)
    mean_copy.start()

    cnt = jax.lax.broadcasted_iota(jnp.uint32, (1, _D), 1)
    na, nb = _threefry(jnp.zeros((1, _D), jnp.uint32), cnt, ks0, ks1)
    nbits = na ^ nb
    u01 = _bits_to_unit(((nbits >> 9)).astype(jnp.int32).astype(jnp.float32))
    lo = jnp.float32(-0.99999994)
    un = jnp.maximum(lo, u01 * (jnp.float32(1.0) - lo) + lo)
    z = jnp.float32(1.4142135623730951) * _erfinv_f32(un)

    cov_copy.wait()
    mean_copy.wait()

    lane_row = jax.lax.broadcasted_iota(jnp.int32, (1, _D), 1)
    _PW = 16
    subp = jax.lax.broadcasted_iota(jnp.int32, (_PW + 1, 1), 0)
    lane_p = jax.lax.broadcasted_iota(jnp.int32, (1, _PW), 1)
    oh_sub = jax.lax.broadcasted_iota(jnp.int32, (_D, _PW), 0)
    oh_lane = jax.lax.broadcasted_iota(jnp.int32, (_D, _PW), 1)

    def panel_step(p, y):
        base = _PW * p
        rows = jnp.concatenate(
            [acov[pl.ds(base, _PW), :], z], axis=0)
        ohp = (oh_sub == base + oh_lane).astype(jnp.float32)
        c = jax.lax.dot_general(
            rows, ohp, (((1,), (0,)), ((), ())),
            preferred_element_type=jnp.float32)
        wrows = []
        for t in range(_PW):
            j = base + t
            colv = c[:, t:t + 1]
            pivot = colv[t:t + 1, :]
            zj = colv[_PW:_PW + 1, :]
            rowt = rows[t:t + 1, :]
            rowm = jnp.where(lane_row >= j, rowt, jnp.float32(0.0))
            w = rowm * jax.lax.rsqrt(pivot)
            y = y + w * zj
            wrows.append(w)
            if t < _PW - 1:
                ip = jnp.float32(1.0) / pivot
                upd = jnp.where((subp > t) & (subp < _PW), colv,
                                jnp.float32(0.0))
                rowcm = jnp.where(lane_p >= t, c[t:t + 1, :],
                                  jnp.float32(0.0))
                c = c - upd * (rowcm * ip)
                rows = rows - upd * (rowm * ip)
        wmat = jnp.concatenate(wrows, axis=0)
        u = jax.lax.dot_general(
            wmat, wmat, (((0,), (0,)), ((), ())),
            preferred_element_type=jnp.float32)
        acov[:] = acov[:] - u
        return y

    y = jax.lax.fori_loop(0, _D // _PW, panel_step,
                          jnp.zeros((1, _D), jnp.float32))
    out_ref[:] = mrow[:] + y


@jax.jit
def kernel(means, covs, weights, key_seed):
    kd = jnp.asarray(key_seed, jnp.int32).reshape(1)
    out = pl.pallas_call(
        _body,
        out_shape=jax.ShapeDtypeStruct((1, _D), jnp.float32),
        in_specs=[
            pl.BlockSpec(memory_space=pltpu.MemorySpace.SMEM),
            pl.BlockSpec(memory_space=pltpu.MemorySpace.VMEM),
            pl.BlockSpec(memory_space=pl.ANY),
            pl.BlockSpec(memory_space=pl.ANY),
        ],
        out_specs=pl.BlockSpec(memory_space=pltpu.MemorySpace.VMEM),
        scratch_shapes=[
            pltpu.VMEM((_D, _D), jnp.float32),
            pltpu.VMEM((1, _D), jnp.float32),
            pltpu.SemaphoreType.DMA,
            pltpu.SemaphoreType.DMA,
        ],
    )(kd, weights.reshape(8, _D), means, covs)
    return out.reshape(_D)


# speedup vs baseline: 1.7445x; 1.0366x over previous
import functools

import jax
import jax.numpy as jnp
from jax.experimental import pallas as pl
from jax.experimental.pallas import tpu as pltpu

_K = 1024
_D = 128
_ROT = ((13, 15, 26, 6), (17, 29, 16, 24))


def _threefry(x0, x1, k0, k1):
    k2 = k0 ^ k1 ^ jnp.uint32(0x1BD11BDA)
    ks = (k0, k1, k2)
    x0 = x0 + k0
    x1 = x1 + k1
    for i in range(5):
        for r in _ROT[i % 2]:
            x0 = x0 + x1
            x1 = (x1 << r) | (x1 >> (32 - r))
            x1 = x1 ^ x0
        x0 = x0 + ks[(i + 1) % 3]
        x1 = x1 + ks[(i + 2) % 3] + jnp.uint32(i + 1)
    return x0, x1


def _bits_to_unit(bits_f32_mantissa):
    return bits_f32_mantissa * jnp.float32(2.0 ** -23)


def _erfinv_f32(x):
    w = -jnp.log1p(-x * x)
    w_c = w - jnp.float32(2.5)
    p_c = jnp.float32(2.81022636e-08)
    for c in (3.43273939e-07, -3.5233877e-06, -4.39150654e-06, 0.00021858087,
              -0.00125372503, -0.00417768164, 0.246640727, 1.50140941):
        p_c = jnp.float32(c) + p_c * w_c
    w_t = jnp.sqrt(w) - jnp.float32(3.0)
    p_t = jnp.float32(-0.000200214257)
    for c in (0.000100950558, 0.00134934322, -0.00367342844, 0.00573950773,
              -0.0076224613, 0.00943887047, 1.00167406, 2.83297682):
        p_t = jnp.float32(c) + p_t * w_t
    return jnp.where(w < jnp.float32(5.0), p_c, p_t) * x


def _body(kd_ref, w_ref, means_hbm, covs_hbm, out_ref, acov, mrow, sem_c, sem_m):
    u32 = jnp.uint32
    k0 = u32(0)
    k1 = kd_ref[0].astype(jnp.uint32)

    a0, b0 = _threefry(u32(0), u32(0), k0, k1)
    a1, b1 = _threefry(u32(0), u32(1), k0, k1)
    ki0, ki1 = a0, b0
    ks0, ks1 = a1, b1

    ua, ub = _threefry(u32(0), u32(0), ki0, ki1)
    ubits = ua ^ ub
    u = _bits_to_unit(((ubits >> 9)).astype(jnp.int32).astype(jnp.float32))

    w = w_ref[:]
    s_total = jnp.sum(w)
    p = w / s_total
    lane = jax.lax.broadcasted_iota(jnp.int32, (_D, _D), 0)
    lane_t = jax.lax.broadcasted_iota(jnp.int32, (_D, _D), 1)
    triu = (lane <= lane_t).astype(jnp.float32)
    lanecum = jnp.dot(p, triu, preferred_element_type=jnp.float32)
    rowtot = lanecum[:, _D - 1:_D]
    inc = rowtot
    for d in (1, 2, 4):
        inc = inc + jnp.concatenate(
            [jnp.zeros((d, 1), jnp.float32), inc[:-d]], axis=0)
    offs = jnp.concatenate(
        [jnp.zeros((1, 1), jnp.float32), inc[:-1]], axis=0)
    p_cuml = offs + lanecum
    r = jnp.max(p_cuml) * (jnp.float32(1.0) - u)
    idx = jnp.sum((p_cuml < r).astype(jnp.int32))

    cov_copy = pltpu.make_async_copy(covs_hbm.at[idx], acov, sem_c)
    cov_copy.start()
    mean_copy = pltpu.make_async_copy(means_hbm.at[pl.ds(idx, 1), :], mrow, sem_m)
    mean_copy.start()

    cnt = jax.lax.broadcasted_iota(jnp.uint32, (1, _D), 1)
    na, nb = _threefry(jnp.zeros((1, _D), jnp.uint32), cnt, ks0, ks1)
    nbits = na ^ nb
    u01 = _bits_to_unit(((nbits >> 9)).astype(jnp.int32).astype(jnp.float32))
    lo = jnp.float32(-0.99999994)
    un = jnp.maximum(lo, u01 * (jnp.float32(1.0) - lo) + lo)
    z = jnp.float32(1.4142135623730951) * _erfinv_f32(un)

    cov_copy.wait()
    mean_copy.wait()

    lane_row = jax.lax.broadcasted_iota(jnp.int32, (1, _D), 1)
    _PW = 32
    subp = jax.lax.broadcasted_iota(jnp.int32, (_PW + 1, 1), 0)
    lane_p = jax.lax.broadcasted_iota(jnp.int32, (1, _PW), 1)
    oh_sub = jax.lax.broadcasted_iota(jnp.int32, (_D, _PW), 0)
    oh_lane = jax.lax.broadcasted_iota(jnp.int32, (_D, _PW), 1)

    def panel_step(p, y):
        base = _PW * p
        rows = jnp.concatenate(
            [acov[pl.ds(base, _PW), :], z], axis=0)
        ohp = (oh_sub == base + oh_lane).astype(jnp.float32)
        c = jax.lax.dot_general(
            rows, ohp, (((1,), (0,)), ((), ())),
            preferred_element_type=jnp.float32)
        wrows = []
        for t in range(_PW):
            j = base + t
            colv = c[:, t:t + 1]
            pivot = colv[t:t + 1, :]
            zj = colv[_PW:_PW + 1, :]
            rowt = rows[t:t + 1, :]
            rowm = jnp.where(lane_row >= j, rowt, jnp.float32(0.0))
            w = rowm * jax.lax.rsqrt(pivot)
            y = y + w * zj
            wrows.append(w)
            if t < _PW - 1:
                ip = jnp.float32(1.0) / pivot
                upd = jnp.where((subp > t) & (subp < _PW), colv,
                                jnp.float32(0.0))
                rowcm = jnp.where(lane_p >= t, c[t:t + 1, :],
                                  jnp.float32(0.0))
                c = c - upd * (rowcm * ip)
                rows = rows - upd * (rowm * ip)
        wmat = jnp.concatenate(wrows, axis=0)
        u = jax.lax.dot_general(
            wmat, wmat, (((0,), (0,)), ((), ())),
            preferred_element_type=jnp.float32)
        acov[:] = acov[:] - u
        return y

    y = jax.lax.fori_loop(0, _D // _PW, panel_step,
                          jnp.zeros((1, _D), jnp.float32))
    out_ref[:] = mrow[:] + y


@jax.jit
def kernel(means, covs, weights, key_seed):
    kd = jnp.asarray(key_seed, jnp.int32).reshape(1)
    out = pl.pallas_call(
        _body,
        out_shape=jax.ShapeDtypeStruct((1, _D), jnp.float32),
        in_specs=[
            pl.BlockSpec(memory_space=pltpu.MemorySpace.SMEM),
            pl.BlockSpec(memory_space=pltpu.MemorySpace.VMEM),
            pl.BlockSpec(memory_space=pl.ANY),
            pl.BlockSpec(memory_space=pl.ANY),
        ],
        out_specs=pl.BlockSpec(memory_space=pltpu.MemorySpace.VMEM),
        scratch_shapes=[
            pltpu.VMEM((_D, _D), jnp.float32),
            pltpu.VMEM((1, _D), jnp.float32),
            pltpu.SemaphoreType.DMA,
            pltpu.SemaphoreType.DMA,
        ],
    )(kd, weights.reshape(8, _D), means, covs)
    return out.reshape(_D)
